# Initial kernel scaffold; baseline (speedup 1.0000x reference)
import jax
import jax.numpy as jnp
from jax.experimental import pallas as pl


def kernel(customer_ids, ages, emb_table, bn_gamma, bn_beta, W, b):
    raise NotImplementedError("write your pallas kernel here")


# speedup vs baseline: 1.3979x; 1.3979x over previous
import functools

import jax
import jax.numpy as jnp
from jax import lax
from jax.experimental import pallas as pl
from jax.experimental.pallas import tpu as pltpu
from jax.experimental.pallas import tpu_sc as plsc

B = 16384
V = 1000
VPAD = 1024
D = 16
OUT = 10
EPS = 1e-5

NC = 2
NS = 16
NW = NC * NS
BPW = B // NW
LANES = 16
CHUNKS = BPW // LANES
GCH = 128
NG = BPW // GCH


def _prep_body(ages_ref, table_ref, wt_ref, bp_ref, g_ref, be_ref, s_ref, p_ref):
    a = ages_ref[...]
    mean = jnp.mean(a)
    var = jnp.mean((a - mean) ** 2)
    inv = lax.rsqrt(var + EPS) * g_ref[0, 0]
    s_ref[...] = jnp.maximum((a - mean) * inv + be_ref[0, 0], 0.0)
    t = jnp.maximum(table_ref[...], 0.0)
    p_ref[...] = jnp.dot(t, wt_ref[...], preferred_element_type=jnp.float32) + bp_ref[...]


_prep = pl.pallas_call(
    _prep_body,
    out_shape=(
        jax.ShapeDtypeStruct((128, 128), jnp.float32),
        jax.ShapeDtypeStruct((VPAD, D), jnp.float32),
    ),
)


def _sc_body(ids_hbm, s_hbm, p_hbm, w16_hbm, out_hbm,
             idx_v, rows_v, s_v, out_v, w16_v, sem):
    wid = lax.axis_index("s") * NC + lax.axis_index("c")
    base = wid * BPW
    pltpu.sync_copy(ids_hbm.at[pl.ds(wid * NG, NG)], idx_v)
    copies = [pltpu.async_copy(p_hbm.at[idx_v.at[r]],
                               rows_v.at[pl.ds(r * GCH, GCH)], sem)
              for r in range(NG)]
    pltpu.sync_copy(s_hbm.at[pl.ds(base, BPW)], s_v)
    pltpu.sync_copy(w16_hbm, w16_v)
    for c in copies:
        c.wait()
    w16 = w16_v[...]

    def chunk(ci, carry):
        i_vec = ci * LANES + lax.iota(jnp.int32, LANES)
        s_chunk = s_v[pl.ds(ci * LANES, LANES)]
        for j in range(OUT):
            jv = jnp.full((LANES,), j, jnp.int32)
            g = plsc.load_gather(rows_v, [i_vec, jv])
            plsc.store_scatter(out_v, [i_vec, jv], g + s_chunk * w16[j])
        return carry

    lax.fori_loop(0, CHUNKS, chunk, 0, unroll=False)
    pltpu.sync_copy(out_v, out_hbm.at[pl.ds(base, BPW)])


_sc_main = functools.partial(
    pl.kernel,
    mesh=plsc.VectorSubcoreMesh(core_axis_name="c", subcore_axis_name="s"),
    out_type=jax.ShapeDtypeStruct((B, OUT), jnp.float32),
    compiler_params=pltpu.CompilerParams(needs_layout_passes=False,
                                         use_tc_tiling_on_sc=False),
    scratch_types=[
        pltpu.VMEM((NG, GCH), jnp.int32),
        pltpu.VMEM((BPW, D), jnp.float32),
        pltpu.VMEM((BPW,), jnp.float32),
        pltpu.VMEM((BPW, OUT), jnp.float32),
        pltpu.VMEM((LANES,), jnp.float32),
        pltpu.SemaphoreType.DMA,
    ],
)(_sc_body)


def kernel(customer_ids, ages, emb_table, bn_gamma, bn_beta, W, b):
    table_p = jnp.zeros((VPAD, D), jnp.float32).at[:V].set(emb_table)
    wt = jnp.zeros((D, D), jnp.float32).at[:, :OUT].set(W[:, :D].T)
    bp = jnp.zeros((1, D), jnp.float32).at[0, :OUT].set(b)
    w16 = jnp.zeros((LANES,), jnp.float32).at[:OUT].set(W[:, D])
    s2, P = _prep(ages.reshape(128, 128), table_p, wt, bp,
                  bn_gamma.reshape(1, 1), bn_beta.reshape(1, 1))
    ids2 = customer_ids.astype(jnp.int32).reshape(NW * NG, GCH)
    return _sc_main(ids2, s2.reshape(B), P, w16)


# speedup vs baseline: 1.4339x; 1.0257x over previous
import functools

import jax
import jax.numpy as jnp
from jax import lax
from jax.experimental import pallas as pl
from jax.experimental.pallas import tpu as pltpu
from jax.experimental.pallas import tpu_sc as plsc

B = 16384
V = 1000
D = 16
OUT = 10
EPS = 1e-5

NC = 2
NS = 16
NW = NC * NS
BPW = B // NW
LANES = 16
CHUNKS = BPW // LANES
GCH = 128
NG = BPW // GCH
APT = B // NS
ROWS = 64


def _splat16(x):
    return jnp.full((LANES,), x, jnp.float32)


def _body(ids_hbm, ages_hbm, table_hbm, gamma_hbm, beta_hbm, w_hbm, b_hbm,
          out_hbm, p_hbm,
          ids_v, a_v, ab_v, t_v, w_v, b_v, g_v, be_v, p_loc, rows_v, out_v,
          st_v, allst_v, shst, sem):
    cid = lax.axis_index("c")
    sid = lax.axis_index("s")
    wid = sid * NC + cid
    base = wid * BPW
    iota = lax.iota(jnp.int32, LANES)

    pltpu.sync_copy(ages_hbm.at[pl.ds(sid * APT, APT)], a_v)
    pltpu.sync_copy(ages_hbm.at[pl.ds(base, BPW)], ab_v)
    for r in range(NG):
        pltpu.sync_copy(ids_hbm.at[pl.ds(base + r * GCH, GCH)], ids_v.at[r])
    base_t = jnp.minimum(sid * ROWS, V - ROWS)
    pltpu.sync_copy(table_hbm.at[pl.ds(base_t, ROWS)], t_v)
    pltpu.sync_copy(w_hbm, w_v)
    pltpu.sync_copy(b_hbm, b_v.at[pl.ds(8, OUT)])
    pltpu.sync_copy(gamma_hbm, g_v.at[pl.ds(8, 1)])
    pltpu.sync_copy(beta_hbm, be_v.at[pl.ds(8, 1)])

    def stat_step(i, carry):
        s1, s2 = carry
        v = a_v[pl.ds(i * LANES, LANES)]
        return s1 + v, s2 + v * v

    z16 = jnp.zeros((LANES,), jnp.float32)
    s1, s2 = lax.fori_loop(0, APT // LANES, stat_step, (z16, z16))
    st_v[pl.ds(0, LANES)] = s1
    st_v[pl.ds(LANES, LANES)] = s2
    pltpu.sync_copy(st_v, shst.at[pl.ds(sid * 2 * LANES, 2 * LANES)])

    wrows = [plsc.load_gather(w_v, [jnp.full((LANES,), j, jnp.int32), iota])
             for j in range(OUT)]
    w16 = [plsc.load_gather(w_v, [jnp.full((LANES,), j, jnp.int32),
                                  jnp.full((LANES,), D, jnp.int32)])
           for j in range(OUT)]
    bspl = [plsc.load_gather(b_v, [jnp.full((LANES,), 8 + j, jnp.int32)])
            for j in range(OUT)]
    for ch in range(ROWS // LANES):
        v_loc = ch * LANES + iota
        feats = [jnp.maximum(
            plsc.load_gather(t_v, [v_loc, jnp.full((LANES,), d, jnp.int32)]),
            0.0) for d in range(D)]
        for j in range(OUT):
            acc = bspl[j]
            for d in range(D):
                acc = acc + feats[d] * wrows[j][d]
            plsc.store_scatter(p_loc, [v_loc, jnp.full((LANES,), j, jnp.int32)],
                               acc)

    @pl.when(sid < NS - 1)
    def _():
        pltpu.sync_copy(p_loc, p_hbm.at[pl.ds(sid * ROWS, ROWS)])

    @pl.when(sid == NS - 1)
    def _():
        off = NS * ROWS - V
        pltpu.sync_copy(p_loc.at[pl.ds(off, ROWS - off)],
                        p_hbm.at[pl.ds((NS - 1) * ROWS, ROWS - off)])

    plsc.subcore_barrier()

    pltpu.sync_copy(shst, allst_v)

    def red_step(i, carry):
        s1, s2 = carry
        return (s1 + allst_v[pl.ds(i * 2 * LANES, LANES)],
                s2 + allst_v[pl.ds(i * 2 * LANES + LANES, LANES)])

    r1, r2 = lax.fori_loop(0, NS, red_step, (z16, z16))
    meanv = _splat16(jnp.sum(r1)) * (1.0 / B)
    varv = _splat16(jnp.sum(r2)) * (1.0 / B) - meanv * meanv
    xv = varv + EPS
    yv = plsc.bitcast(0x5F3759DF - (plsc.bitcast(xv, jnp.int32) >> 1),
                      jnp.float32)
    for _ in range(4):
        yv = yv * (1.5 - 0.5 * xv * yv * yv)
    eight_i = jnp.full((LANES,), 8, jnp.int32)
    gspl = plsc.load_gather(g_v, [eight_i])
    bespl = plsc.load_gather(be_v, [eight_i])
    k1 = yv * gspl

    copies = [pltpu.async_copy(p_hbm.at[ids_v.at[r]],
                               rows_v.at[pl.ds(r * GCH, GCH)], sem)
              for r in range(NG)]
    for c in copies:
        c.wait()

    def chunk(ci, carry):
        i_vec = ci * LANES + iota
        a_chunk = ab_v[pl.ds(ci * LANES, LANES)]
        s_chunk = jnp.maximum((a_chunk - meanv) * k1 + bespl, 0.0)
        for j in range(OUT):
            jv = jnp.full((LANES,), j, jnp.int32)
            g = plsc.load_gather(rows_v, [i_vec, jv])
            plsc.store_scatter(out_v, [i_vec, jv], g + s_chunk * w16[j])
        return carry

    lax.fori_loop(0, CHUNKS, chunk, 0)
    pltpu.sync_copy(out_v, out_hbm.at[pl.ds(base, BPW)])


_sc_kernel = functools.partial(
    pl.kernel,
    mesh=plsc.VectorSubcoreMesh(core_axis_name="c", subcore_axis_name="s"),
    out_type=(jax.ShapeDtypeStruct((B, OUT), jnp.float32),
              jax.ShapeDtypeStruct((V, D), jnp.float32)),
    compiler_params=pltpu.CompilerParams(needs_layout_passes=False,
                                         use_tc_tiling_on_sc=False),
    scratch_types=[
        pltpu.VMEM((NG, GCH), jnp.int32),
        pltpu.VMEM((APT,), jnp.float32),
        pltpu.VMEM((BPW,), jnp.float32),
        pltpu.VMEM((ROWS, D), jnp.float32),
        pltpu.VMEM((OUT, D + 1), jnp.float32),
        pltpu.VMEM((8 + OUT,), jnp.float32),
        pltpu.VMEM((16,), jnp.float32),
        pltpu.VMEM((16,), jnp.float32),
        pltpu.VMEM((ROWS, D), jnp.float32),
        pltpu.VMEM((BPW, D), jnp.float32),
        pltpu.VMEM((BPW, OUT), jnp.float32),
        pltpu.VMEM((2 * LANES,), jnp.float32),
        pltpu.VMEM((NS * 2 * LANES,), jnp.float32),
        pltpu.VMEM_SHARED((NS * 2 * LANES,), jnp.float32),
        pltpu.SemaphoreType.DMA,
    ],
)(_body)


def kernel(customer_ids, ages, emb_table, bn_gamma, bn_beta, W, b):
    out, _ = _sc_kernel(customer_ids, ages, emb_table, bn_gamma, bn_beta, W, b)
    return out


# speedup vs baseline: 1.6168x; 1.1275x over previous
import functools

import jax
import jax.numpy as jnp
from jax import lax
from jax.experimental import pallas as pl
from jax.experimental.pallas import tpu as pltpu
from jax.experimental.pallas import tpu_sc as plsc

B = 16384
V = 1000
D = 16
OUT = 10
EPS = 1e-5

NC = 2
NS = 16
NW = NC * NS
BPW = B // NW
LANES = 16
CHUNKS = BPW // LANES
GCH = 128
NG = BPW // GCH
APT = B // NS
ROWS = 64


def _splat16(x):
    return jnp.full((LANES,), x, jnp.float32)


def _body(ids_hbm, ages_hbm, table_hbm, gamma_hbm, beta_hbm, w_hbm, b_hbm,
          out_hbm, p_hbm,
          ids_v, a_v, ab_v, t_v, w_v, b_v, g_v, be_v, p_loc, rows_v, out_v,
          st_v, allst_v, shst, sem):
    cid = lax.axis_index("c")
    sid = lax.axis_index("s")
    wid = sid * NC + cid
    base = wid * BPW
    iota = lax.iota(jnp.int32, LANES)

    base_t = jnp.minimum(sid * ROWS, V - ROWS)
    stage = [
        pltpu.async_copy(ages_hbm.at[pl.ds(sid * APT, APT)], a_v, sem),
        pltpu.async_copy(ages_hbm.at[pl.ds(base, BPW)], ab_v, sem),
        pltpu.async_copy(table_hbm.at[pl.ds(base_t, ROWS)], t_v, sem),
        pltpu.async_copy(w_hbm, w_v, sem),
        pltpu.async_copy(b_hbm, b_v.at[pl.ds(8, OUT)], sem),
        pltpu.async_copy(gamma_hbm, g_v.at[pl.ds(8, 1)], sem),
        pltpu.async_copy(beta_hbm, be_v.at[pl.ds(8, 1)], sem),
    ] + [pltpu.async_copy(ids_hbm.at[pl.ds(base + r * GCH, GCH)], ids_v.at[r],
                          sem)
         for r in range(NG)]
    for c in stage:
        c.wait()

    def stat_step(i, carry):
        s1, s2 = carry
        v = a_v[pl.ds(i * LANES, LANES)]
        return s1 + v, s2 + v * v

    z16 = jnp.zeros((LANES,), jnp.float32)
    s1, s2 = lax.fori_loop(0, APT // LANES, stat_step, (z16, z16))
    st_v[pl.ds(0, LANES)] = s1
    st_v[pl.ds(LANES, LANES)] = s2
    pltpu.sync_copy(st_v, shst.at[pl.ds(sid * 2 * LANES, 2 * LANES)])

    wrows = [plsc.load_gather(w_v, [jnp.full((LANES,), j, jnp.int32), iota])
             for j in range(OUT)]
    w16 = [plsc.load_gather(w_v, [jnp.full((LANES,), j, jnp.int32),
                                  jnp.full((LANES,), D, jnp.int32)])
           for j in range(OUT)]
    bspl = [plsc.load_gather(b_v, [jnp.full((LANES,), 8 + j, jnp.int32)])
            for j in range(OUT)]
    def p_chunk(ch, carry):
        v_loc = ch * LANES + iota
        feats = [jnp.maximum(
            plsc.load_gather(t_v, [v_loc, jnp.full((LANES,), d, jnp.int32)]),
            0.0) for d in range(D)]
        for j in range(OUT):
            acc = bspl[j]
            for d in range(D):
                acc = acc + feats[d] * wrows[j][d]
            plsc.store_scatter(p_loc, [v_loc, jnp.full((LANES,), j, jnp.int32)],
                               acc)
        return carry

    lax.fori_loop(0, ROWS // LANES, p_chunk, 0)

    @pl.when(sid < NS - 1)
    def _():
        pltpu.sync_copy(p_loc, p_hbm.at[pl.ds(sid * ROWS, ROWS)])

    @pl.when(sid == NS - 1)
    def _():
        off = NS * ROWS - V
        pltpu.sync_copy(p_loc.at[pl.ds(off, ROWS - off)],
                        p_hbm.at[pl.ds((NS - 1) * ROWS, ROWS - off)])

    plsc.subcore_barrier()

    pltpu.sync_copy(shst, allst_v)

    def red_step(i, carry):
        s1, s2 = carry
        return (s1 + allst_v[pl.ds(i * 2 * LANES, LANES)],
                s2 + allst_v[pl.ds(i * 2 * LANES + LANES, LANES)])

    r1, r2 = lax.fori_loop(0, NS, red_step, (z16, z16))
    meanv = _splat16(jnp.sum(r1)) * (1.0 / B)
    varv = _splat16(jnp.sum(r2)) * (1.0 / B) - meanv * meanv
    xv = varv + EPS
    yv = plsc.bitcast(0x5F3759DF - (plsc.bitcast(xv, jnp.int32) >> 1),
                      jnp.float32)
    for _ in range(4):
        yv = yv * (1.5 - 0.5 * xv * yv * yv)
    eight_i = jnp.full((LANES,), 8, jnp.int32)
    gspl = plsc.load_gather(g_v, [eight_i])
    bespl = plsc.load_gather(be_v, [eight_i])
    k1 = yv * gspl

    copies = [pltpu.async_copy(p_hbm.at[ids_v.at[r]],
                               rows_v.at[pl.ds(r * GCH, GCH)], sem)
              for r in range(NG)]
    for c in copies:
        c.wait()

    def chunk(ci, carry):
        i_vec = ci * LANES + iota
        a_chunk = ab_v[pl.ds(ci * LANES, LANES)]
        s_chunk = jnp.maximum((a_chunk - meanv) * k1 + bespl, 0.0)
        for j in range(OUT):
            jv = jnp.full((LANES,), j, jnp.int32)
            g = plsc.load_gather(rows_v, [i_vec, jv])
            plsc.store_scatter(out_v, [i_vec, jv], g + s_chunk * w16[j])
        return carry

    lax.fori_loop(0, CHUNKS, chunk, 0)
    pltpu.sync_copy(out_v, out_hbm.at[pl.ds(base, BPW)])


_sc_kernel = functools.partial(
    pl.kernel,
    mesh=plsc.VectorSubcoreMesh(core_axis_name="c", subcore_axis_name="s"),
    out_type=(jax.ShapeDtypeStruct((B, OUT), jnp.float32),
              jax.ShapeDtypeStruct((V, D), jnp.float32)),
    compiler_params=pltpu.CompilerParams(needs_layout_passes=False,
                                         use_tc_tiling_on_sc=False),
    scratch_types=[
        pltpu.VMEM((NG, GCH), jnp.int32),
        pltpu.VMEM((APT,), jnp.float32),
        pltpu.VMEM((BPW,), jnp.float32),
        pltpu.VMEM((ROWS, D), jnp.float32),
        pltpu.VMEM((OUT, D + 1), jnp.float32),
        pltpu.VMEM((8 + OUT,), jnp.float32),
        pltpu.VMEM((16,), jnp.float32),
        pltpu.VMEM((16,), jnp.float32),
        pltpu.VMEM((ROWS, D), jnp.float32),
        pltpu.VMEM((BPW, D), jnp.float32),
        pltpu.VMEM((BPW, OUT), jnp.float32),
        pltpu.VMEM((2 * LANES,), jnp.float32),
        pltpu.VMEM((NS * 2 * LANES,), jnp.float32),
        pltpu.VMEM_SHARED((NS * 2 * LANES,), jnp.float32),
        pltpu.SemaphoreType.DMA,
    ],
)(_body)


def kernel(customer_ids, ages, emb_table, bn_gamma, bn_beta, W, b):
    out, _ = _sc_kernel(customer_ids, ages, emb_table, bn_gamma, bn_beta, W, b)
    return out
